# baseline (speedup 1.0000x reference)
import functools

import jax
import jax.numpy as jnp
from jax import lax
from jax.experimental import pallas as pl
from jax.experimental.pallas import tpu as pltpu
from jax.experimental.pallas import tpu_sc as plsc

N_NODES = 10000
N_EDGES = 320000
D_IN = 128
D_HID = 64
N_CLS = 6

NC, NS = 2, 16
NW = NC * NS
CHUNK = 125
ROWS_PER_TILE = N_NODES // NS
EW = N_EDGES // NW
NSTREAM = EW // CHUNK
DEGW = 16
NBUF = 2
ZROWS = 125
ZITER = ROWS_PER_TILE // ZROWS


def _deg_body(ei_hbm, ones_hbm, zeros_hbm, out_hbm, dst_v, ones_v, zbuf_v,
              semd, deg_sh):
    c = lax.axis_index("c")
    s = lax.axis_index("s")
    wid = s * NC + c
    pltpu.sync_copy(zeros_hbm, zbuf_v)
    pltpu.sync_copy(zbuf_v, deg_sh.at[pl.ds(s * ROWS_PER_TILE, ROWS_PER_TILE)])
    pltpu.sync_copy(ones_hbm, ones_v)
    pltpu.sync_copy(ei_hbm.at[1, wid], dst_v)
    plsc.subcore_barrier()

    def body(j, carry):
        pltpu.async_copy(ones_v, deg_sh.at[dst_v.at[j]], semd, add=True)
        return carry

    lax.fori_loop(0, NSTREAM, body, 0)

    def drain(j, carry):
        pltpu.make_async_copy(ones_v, deg_sh.at[dst_v.at[j]], semd).wait()
        return carry

    lax.fori_loop(0, NSTREAM, drain, 0)
    plsc.subcore_barrier()
    pltpu.sync_copy(deg_sh.at[pl.ds(s * ROWS_PER_TILE, ROWS_PER_TILE)], zbuf_v)
    pltpu.sync_copy(zbuf_v, out_hbm.at[c, pl.ds(s * ROWS_PER_TILE, ROWS_PER_TILE)])


_deg_call = pl.kernel(
    _deg_body,
    out_type=jax.ShapeDtypeStruct((NC, N_NODES, DEGW), jnp.float32),
    mesh=plsc.VectorSubcoreMesh(core_axis_name="c", subcore_axis_name="s"),
    compiler_params=pltpu.CompilerParams(use_tc_tiling_on_sc=False),
    scratch_types=[
        pltpu.VMEM((NSTREAM, CHUNK), jnp.int32),
        pltpu.VMEM((CHUNK, DEGW), jnp.float32),
        pltpu.VMEM((ROWS_PER_TILE, DEGW), jnp.float32),
        pltpu.SemaphoreType.DMA,
        pltpu.VMEM_SHARED((N_NODES, DEGW), jnp.float32),
    ],
)


def _agg_body(y_hbm, ei_hbm, zeros_hbm, out_hbm,
              src_v, dst_v, rows_v, zbuf_v, semg, acc_sh, ytab_sh):
    c = lax.axis_index("c")
    s = lax.axis_index("s")
    wid = s * NC + c
    tsl = pl.ds(s * ROWS_PER_TILE, ROWS_PER_TILE)
    pltpu.sync_copy(y_hbm.at[tsl], ytab_sh.at[tsl])
    pltpu.sync_copy(zeros_hbm, zbuf_v)
    for t in range(ZITER):
        pltpu.sync_copy(zbuf_v,
                        acc_sh.at[pl.ds(s * ROWS_PER_TILE + t * ZROWS, ZROWS)])
    pltpu.sync_copy(ei_hbm.at[0, wid], src_v)
    pltpu.sync_copy(ei_hbm.at[1, wid], dst_v)
    plsc.subcore_barrier()

    def _fire(idx, ns):
        @pl.when(lax.rem(idx, 4) == 0)
        def _():
            pltpu.async_copy(y_hbm.at[src_v.at[idx]], rows_v.at[ns],
                             semg.at[ns])

        @pl.when(lax.rem(idx, 4) != 0)
        def _():
            pltpu.async_copy(ytab_sh.at[src_v.at[idx]], rows_v.at[ns],
                             semg.at[ns])

    def _wait(idx, slot):
        @pl.when(lax.rem(idx, 4) == 0)
        def _():
            pltpu.make_async_copy(y_hbm.at[src_v.at[idx]], rows_v.at[slot],
                                  semg.at[slot]).wait()

        @pl.when(lax.rem(idx, 4) != 0)
        def _():
            pltpu.make_async_copy(ytab_sh.at[src_v.at[idx]], rows_v.at[slot],
                                  semg.at[slot]).wait()

    for p in range(NBUF - 1):
        _fire(jnp.int32(p), p)

    def body(j, carry):
        slot = lax.rem(j, NBUF)

        @pl.when(j + NBUF - 1 < NSTREAM)
        def _():
            _fire(j + NBUF - 1, lax.rem(j + NBUF - 1, NBUF))

        _wait(j, slot)
        pltpu.sync_copy(rows_v.at[slot], acc_sh.at[dst_v.at[j]], add=True)
        return carry

    lax.fori_loop(0, NSTREAM, body, 0)
    plsc.subcore_barrier()
    for t in range(ZITER):
        sl = pl.ds(s * ROWS_PER_TILE + t * ZROWS, ZROWS)
        pltpu.sync_copy(acc_sh.at[sl], zbuf_v)
        pltpu.sync_copy(zbuf_v, out_hbm.at[c, sl])


_agg_call = pl.kernel(
    _agg_body,
    out_type=jax.ShapeDtypeStruct((NC, N_NODES, D_HID), jnp.float32),
    mesh=plsc.VectorSubcoreMesh(core_axis_name="c", subcore_axis_name="s"),
    compiler_params=pltpu.CompilerParams(use_tc_tiling_on_sc=False),
    scratch_types=[
        pltpu.VMEM((NSTREAM, CHUNK), jnp.int32),
        pltpu.VMEM((NSTREAM, CHUNK), jnp.int32),
        pltpu.VMEM((NBUF, CHUNK, D_HID), jnp.float32),
        pltpu.VMEM((ZROWS, D_HID), jnp.float32),
        pltpu.SemaphoreType.DMA((NBUF,)),
        pltpu.VMEM_SHARED((N_NODES, D_HID), jnp.float32),
        pltpu.VMEM_SHARED((N_NODES, D_HID), jnp.float32),
    ],
)

R_BLK = 2000
GRID = N_NODES // R_BLK


def _dinv_of(degp_ref):
    deg = degp_ref[0, :, 0:1] + degp_ref[1, :, 0:1] + 1.0
    return lax.rsqrt(deg)


def _tc_matmul(x_ref, w_ref, y_ref):
    y_ref[...] = jnp.dot(x_ref[...], w_ref[...],
                         preferred_element_type=jnp.float32)


def _tc_scale(xw_ref, degp_ref, y_ref):
    y_ref[...] = xw_ref[...] * _dinv_of(degp_ref)


def _tc_mid(sp_ref, y_ref, degp_ref, w_ref, b_ref, out_ref):
    dinv = _dinv_of(degp_ref)
    agg = sp_ref[0] + sp_ref[1] + y_ref[...]
    h = jax.nn.relu(dinv * agg + b_ref[...])
    out_ref[...] = jnp.dot(h, w_ref[...],
                           preferred_element_type=jnp.float32) * dinv


def _log_softmax(z):
    z = z - jnp.max(z, axis=1, keepdims=True)
    return z - jnp.log(jnp.sum(jnp.exp(z), axis=1, keepdims=True))


def _tc_head(sp_ref, y_ref, degp_ref, b_ref, wi_ref, bi_ref, wl_ref, bl_ref,
             imp_ref, lik_ref):
    dinv = _dinv_of(degp_ref)
    agg = sp_ref[0] + sp_ref[1] + y_ref[...]
    h = jax.nn.relu(dinv * agg + b_ref[...])
    imp_ref[...] = _log_softmax(
        jnp.dot(h, wi_ref[...], preferred_element_type=jnp.float32) + bi_ref[...])
    lik_ref[...] = _log_softmax(
        jnp.dot(h, wl_ref[...], preferred_element_type=jnp.float32) + bl_ref[...])


def _row_spec(width):
    return pl.BlockSpec((R_BLK, width), lambda i: (i, 0))


_DEGP_SPEC = pl.BlockSpec((NC, R_BLK, DEGW), lambda i: (0, i, 0))
_SP_SPEC = pl.BlockSpec((NC, R_BLK, D_HID), lambda i: (0, i, 0))


def _full(shape):
    nd = len(shape)
    return pl.BlockSpec(shape, lambda i: (0,) * nd)


def kernel(x, edge_index, W1, b1, W2, b2, Wi, bi, Wl, bl):
    ei = edge_index.astype(jnp.int32).reshape(2, NW, NSTREAM, CHUNK)
    zeros64 = jnp.zeros((ZROWS, D_HID), jnp.float32)
    zeros16 = jnp.zeros((ROWS_PER_TILE, DEGW), jnp.float32)
    ones16 = jnp.ones((CHUNK, DEGW), jnp.float32)
    b1r = b1.reshape(1, D_HID)
    b2r = b2.reshape(1, D_HID)
    bir = bi.reshape(1, N_CLS)
    blr = bl.reshape(1, N_CLS)

    degp = _deg_call(ei, ones16, zeros16)

    xw = pl.pallas_call(
        _tc_matmul,
        grid=(GRID,),
        in_specs=[_row_spec(D_IN), _full((D_IN, D_HID))],
        out_specs=_row_spec(D_HID),
        out_shape=jax.ShapeDtypeStruct((N_NODES, D_HID), jnp.float32),
    )(x, W1)

    y1 = pl.pallas_call(
        _tc_scale,
        grid=(GRID,),
        in_specs=[_row_spec(D_HID), _DEGP_SPEC],
        out_specs=_row_spec(D_HID),
        out_shape=jax.ShapeDtypeStruct((N_NODES, D_HID), jnp.float32),
    )(xw, degp)

    s1 = _agg_call(y1, ei, zeros64)

    y2 = pl.pallas_call(
        _tc_mid,
        grid=(GRID,),
        in_specs=[_SP_SPEC, _row_spec(D_HID), _DEGP_SPEC,
                  _full((D_HID, D_HID)), _full((1, D_HID))],
        out_specs=_row_spec(D_HID),
        out_shape=jax.ShapeDtypeStruct((N_NODES, D_HID), jnp.float32),
    )(s1, y1, degp, W2, b2r)

    s2 = _agg_call(y2, ei, zeros64)

    impact, likelihood = pl.pallas_call(
        _tc_head,
        grid=(GRID,),
        in_specs=[_SP_SPEC, _row_spec(D_HID), _DEGP_SPEC, _full((1, D_HID)),
                  _full((D_HID, N_CLS)), _full((1, N_CLS)),
                  _full((D_HID, N_CLS)), _full((1, N_CLS))],
        out_specs=[_row_spec(N_CLS), _row_spec(N_CLS)],
        out_shape=[jax.ShapeDtypeStruct((N_NODES, N_CLS), jnp.float32),
                   jax.ShapeDtypeStruct((N_NODES, N_CLS), jnp.float32)],
    )(s2, y2, degp, b2r, Wi, bir, Wl, blr)

    return (impact, likelihood)


# speedup vs baseline: 1.0429x; 1.0429x over previous
import jax
import jax.numpy as jnp
from jax import lax
from jax.experimental import pallas as pl
from jax.experimental.pallas import tpu as pltpu
from jax.experimental.pallas import tpu_sc as plsc

N_NODES = 10000
N_EDGES = 320000
D_IN = 128
D_HID = 64
N_CLS = 6

NC, NS = 2, 16
NW = NC * NS
CHUNK = 125
ROWS_PER_TILE = N_NODES // NS
EW = N_EDGES // NW
NSTREAM = EW // CHUNK
DEGW = 16
NBUF = 2
ZROWS = 125
ZITER = ROWS_PER_TILE // ZROWS


def _deg_body(ei_hbm, ones_hbm, zeros_hbm, out_hbm, dst_v, ones_v, zbuf_v,
              semd, deg_sh):
    c = lax.axis_index("c")
    s = lax.axis_index("s")
    wid = s * NC + c
    pltpu.sync_copy(zeros_hbm, zbuf_v)
    pltpu.sync_copy(zbuf_v, deg_sh.at[pl.ds(s * ROWS_PER_TILE, ROWS_PER_TILE)])
    pltpu.sync_copy(ones_hbm, ones_v)
    pltpu.sync_copy(ei_hbm.at[1, wid], dst_v)
    plsc.subcore_barrier()

    def body(j, carry):
        pltpu.async_copy(ones_v, deg_sh.at[dst_v.at[j]], semd, add=True)
        return carry

    lax.fori_loop(0, NSTREAM, body, 0)

    def drain(j, carry):
        pltpu.make_async_copy(ones_v, deg_sh.at[dst_v.at[j]], semd).wait()
        return carry

    lax.fori_loop(0, NSTREAM, drain, 0)
    plsc.subcore_barrier()
    pltpu.sync_copy(deg_sh.at[pl.ds(s * ROWS_PER_TILE, ROWS_PER_TILE)], zbuf_v)
    pltpu.sync_copy(zbuf_v, out_hbm.at[c, pl.ds(s * ROWS_PER_TILE, ROWS_PER_TILE)])


_deg_call = pl.kernel(
    _deg_body,
    out_type=jax.ShapeDtypeStruct((NC, N_NODES, DEGW), jnp.float32),
    mesh=plsc.VectorSubcoreMesh(core_axis_name="c", subcore_axis_name="s"),
    compiler_params=pltpu.CompilerParams(use_tc_tiling_on_sc=False),
    scratch_types=[
        pltpu.VMEM((NSTREAM, CHUNK), jnp.int32),
        pltpu.VMEM((CHUNK, DEGW), jnp.float32),
        pltpu.VMEM((ROWS_PER_TILE, DEGW), jnp.float32),
        pltpu.SemaphoreType.DMA,
        pltpu.VMEM_SHARED((N_NODES, DEGW), jnp.float32),
    ],
)


def _agg_body(y_hbm, ei_hbm, zeros_hbm, out_hbm,
              src_v, dst_v, rows_v, zbuf_v, semg, acc_sh, ytab_sh):
    c = lax.axis_index("c")
    s = lax.axis_index("s")
    wid = s * NC + c
    tsl = pl.ds(s * ROWS_PER_TILE, ROWS_PER_TILE)
    pltpu.sync_copy(y_hbm.at[tsl], ytab_sh.at[tsl])
    pltpu.sync_copy(zeros_hbm, zbuf_v)
    for t in range(ZITER):
        pltpu.sync_copy(zbuf_v,
                        acc_sh.at[pl.ds(s * ROWS_PER_TILE + t * ZROWS, ZROWS)])
    pltpu.sync_copy(ei_hbm.at[0, wid], src_v)
    pltpu.sync_copy(ei_hbm.at[1, wid], dst_v)
    plsc.subcore_barrier()

    for p in range(NBUF - 1):
        pltpu.async_copy(ytab_sh.at[src_v.at[p]], rows_v.at[p], semg.at[p])

    def body(j, carry):
        slot = lax.rem(j, NBUF)

        @pl.when(j + NBUF - 1 < NSTREAM)
        def _():
            ns = lax.rem(j + NBUF - 1, NBUF)
            pltpu.async_copy(ytab_sh.at[src_v.at[j + NBUF - 1]], rows_v.at[ns],
                             semg.at[ns])

        pltpu.make_async_copy(ytab_sh.at[src_v.at[j]], rows_v.at[slot],
                              semg.at[slot]).wait()
        pltpu.sync_copy(rows_v.at[slot], acc_sh.at[dst_v.at[j]], add=True)
        return carry

    lax.fori_loop(0, NSTREAM, body, 0)
    plsc.subcore_barrier()
    for t in range(ZITER):
        sl = pl.ds(s * ROWS_PER_TILE + t * ZROWS, ZROWS)
        pltpu.sync_copy(acc_sh.at[sl], zbuf_v)
        pltpu.sync_copy(zbuf_v, out_hbm.at[c, sl])


_agg_call = pl.kernel(
    _agg_body,
    out_type=jax.ShapeDtypeStruct((NC, N_NODES, D_HID), jnp.float32),
    mesh=plsc.VectorSubcoreMesh(core_axis_name="c", subcore_axis_name="s"),
    compiler_params=pltpu.CompilerParams(use_tc_tiling_on_sc=False),
    scratch_types=[
        pltpu.VMEM((NSTREAM, CHUNK), jnp.int32),
        pltpu.VMEM((NSTREAM, CHUNK), jnp.int32),
        pltpu.VMEM((NBUF, CHUNK, D_HID), jnp.float32),
        pltpu.VMEM((ZROWS, D_HID), jnp.float32),
        pltpu.SemaphoreType.DMA((NBUF,)),
        pltpu.VMEM_SHARED((N_NODES, D_HID), jnp.float32),
        pltpu.VMEM_SHARED((N_NODES, D_HID), jnp.float32),
    ],
)

R_BLK = 2000
GRID = N_NODES // R_BLK


def _dinv_of(degp_ref):
    deg = degp_ref[0, :, 0:1] + degp_ref[1, :, 0:1] + 1.0
    return lax.rsqrt(deg)


def _tc_matmul(x_ref, w_ref, y_ref):
    y_ref[...] = jnp.dot(x_ref[...], w_ref[...],
                         preferred_element_type=jnp.float32)


def _tc_scale(xw_ref, degp_ref, y_ref):
    y_ref[...] = xw_ref[...] * _dinv_of(degp_ref)


def _tc_mid(sp_ref, y_ref, degp_ref, w_ref, b_ref, out_ref):
    dinv = _dinv_of(degp_ref)
    agg = sp_ref[0] + sp_ref[1] + y_ref[...]
    h = jax.nn.relu(dinv * agg + b_ref[...])
    out_ref[...] = jnp.dot(h, w_ref[...],
                           preferred_element_type=jnp.float32) * dinv


def _log_softmax(z):
    z = z - jnp.max(z, axis=1, keepdims=True)
    return z - jnp.log(jnp.sum(jnp.exp(z), axis=1, keepdims=True))


def _tc_head(sp_ref, y_ref, degp_ref, b_ref, wi_ref, bi_ref, wl_ref, bl_ref,
             imp_ref, lik_ref):
    dinv = _dinv_of(degp_ref)
    agg = sp_ref[0] + sp_ref[1] + y_ref[...]
    h = jax.nn.relu(dinv * agg + b_ref[...])
    imp_ref[...] = _log_softmax(
        jnp.dot(h, wi_ref[...], preferred_element_type=jnp.float32) + bi_ref[...])
    lik_ref[...] = _log_softmax(
        jnp.dot(h, wl_ref[...], preferred_element_type=jnp.float32) + bl_ref[...])


def _row_spec(width):
    return pl.BlockSpec((R_BLK, width), lambda i: (i, 0))


_DEGP_SPEC = pl.BlockSpec((NC, R_BLK, DEGW), lambda i: (0, i, 0))
_SP_SPEC = pl.BlockSpec((NC, R_BLK, D_HID), lambda i: (0, i, 0))


def _full(shape):
    nd = len(shape)
    return pl.BlockSpec(shape, lambda i: (0,) * nd)


def kernel(x, edge_index, W1, b1, W2, b2, Wi, bi, Wl, bl):
    ei = edge_index.astype(jnp.int32).reshape(2, NW, NSTREAM, CHUNK)
    zeros64 = jnp.zeros((ZROWS, D_HID), jnp.float32)
    zeros16 = jnp.zeros((ROWS_PER_TILE, DEGW), jnp.float32)
    ones16 = jnp.ones((CHUNK, DEGW), jnp.float32)
    b1r = b1.reshape(1, D_HID)
    b2r = b2.reshape(1, D_HID)
    bir = bi.reshape(1, N_CLS)
    blr = bl.reshape(1, N_CLS)

    degp = _deg_call(ei, ones16, zeros16)

    xw = pl.pallas_call(
        _tc_matmul,
        grid=(GRID,),
        in_specs=[_row_spec(D_IN), _full((D_IN, D_HID))],
        out_specs=_row_spec(D_HID),
        out_shape=jax.ShapeDtypeStruct((N_NODES, D_HID), jnp.float32),
    )(x, W1)

    y1 = pl.pallas_call(
        _tc_scale,
        grid=(GRID,),
        in_specs=[_row_spec(D_HID), _DEGP_SPEC],
        out_specs=_row_spec(D_HID),
        out_shape=jax.ShapeDtypeStruct((N_NODES, D_HID), jnp.float32),
    )(xw, degp)

    s1 = _agg_call(y1, ei, zeros64)

    y2 = pl.pallas_call(
        _tc_mid,
        grid=(GRID,),
        in_specs=[_SP_SPEC, _row_spec(D_HID), _DEGP_SPEC,
                  _full((D_HID, D_HID)), _full((1, D_HID))],
        out_specs=_row_spec(D_HID),
        out_shape=jax.ShapeDtypeStruct((N_NODES, D_HID), jnp.float32),
    )(s1, y1, degp, W2, b2r)

    s2 = _agg_call(y2, ei, zeros64)

    impact, likelihood = pl.pallas_call(
        _tc_head,
        grid=(GRID,),
        in_specs=[_SP_SPEC, _row_spec(D_HID), _DEGP_SPEC, _full((1, D_HID)),
                  _full((D_HID, N_CLS)), _full((1, N_CLS)),
                  _full((D_HID, N_CLS)), _full((1, N_CLS))],
        out_specs=[_row_spec(N_CLS), _row_spec(N_CLS)],
        out_shape=[jax.ShapeDtypeStruct((N_NODES, N_CLS), jnp.float32),
                   jax.ShapeDtypeStruct((N_NODES, N_CLS), jnp.float32)],
    )(s2, y2, degp, b2r, Wi, bir, Wl, blr)

    return (impact, likelihood)
